# baseline (speedup 1.0000x reference)
import functools

import jax
import jax.numpy as jnp
from jax import lax
from jax.experimental import pallas as pl
from jax.experimental.pallas import tpu as pltpu
from jax.experimental.pallas import tpu_sc as plsc

DIM = 64
N_HOPS = 2
N_MEMORY = 20
N_REL = 9
RDIM = 8
BATCH = 1024
KGE_W = 0.01
L2_W = 1e-5

NC = 2
NS = 16
NW = NC * NS
IDXW = 128
SUB = 4
CH = IDXW * SUB
NCHUNK = 42
RPW = CH * NCHUNK
TOT = NW * RPW


def _sc_gather_body(idx_hbm, table_hbm, out_hbm,
                    idx_a, idx_b, rows_a, rows_b,
                    gsem_a, gsem_b, ssem_a, ssem_b):
    cid = lax.axis_index("c")
    sid = lax.axis_index("s")
    wid = sid * NC + cid
    base = wid * RPW
    ibase = wid * (RPW // IDXW)
    idx_bufs = [idx_a, idx_b]
    row_bufs = [rows_a, rows_b]
    gsems = [gsem_a, gsem_b]
    ssems = [ssem_a, ssem_b]

    def fire(b, g):
        pltpu.sync_copy(idx_hbm.at[pl.ds(ibase + g * SUB, SUB)], idx_bufs[b])
        for j in range(SUB):
            pltpu.make_async_copy(
                table_hbm.at[idx_bufs[b].at[j]],
                row_bufs[b].at[pl.ds(j * IDXW, IDXW)],
                gsems[b],
            ).start()

    def gwait(b):
        for j in range(SUB):
            pltpu.make_async_copy(
                table_hbm.at[idx_bufs[b].at[j]],
                row_bufs[b].at[pl.ds(j * IDXW, IDXW)],
                gsems[b],
            ).wait()

    for b in range(2):
        fire(b, b)

    def body(i, carry):
        for b in range(2):
            g = 2 * i + b
            gwait(b)
            st = pltpu.make_async_copy(
                row_bufs[b], out_hbm.at[pl.ds(base + g * CH, CH)], ssems[b])
            st.start()
            st.wait()
            fire(b, g + 2)
        return carry

    lax.fori_loop(0, (NCHUNK - 2) // 2, body, 0)

    for b in range(2):
        g = NCHUNK - 2 + b
        gwait(b)
        pltpu.sync_copy(row_bufs[b], out_hbm.at[pl.ds(base + g * CH, CH)])


@functools.cache
def _sc_gather_kernel():
    return functools.partial(
        pl.kernel,
        out_type=jax.ShapeDtypeStruct((TOT, DIM), jnp.float32),
        mesh=plsc.VectorSubcoreMesh(core_axis_name="c", subcore_axis_name="s"),
        compiler_params=pltpu.CompilerParams(use_tc_tiling_on_sc=False),
        scratch_types=[
            pltpu.VMEM((SUB, IDXW), jnp.int32),
            pltpu.VMEM((SUB, IDXW), jnp.int32),
            pltpu.VMEM((CH, DIM), jnp.float32),
            pltpu.VMEM((CH, DIM), jnp.float32),
            pltpu.SemaphoreType.DMA,
            pltpu.SemaphoreType.DMA,
            pltpu.SemaphoreType.DMA,
            pltpu.SemaphoreType.DMA,
        ],
    )(_sc_gather_body)


def _gather_rows(idx_all, table):
    return _sc_gather_kernel()(idx_all.reshape(TOT // IDXW, IDXW), table)


BB = 16
GSTEPS = BATCH // BB


def _dense_body(items, neg, h0, t0, h1, t1, mr0, mr1,
                rel, tmat, w1, w2, out_ref):
    step = pl.program_id(0)

    @pl.when(step == 0)
    def _init():
        out_ref[0] = 0.0
        out_ref[1] = 0.0
        out_ref[2] = jnp.sum(tmat[...] * tmat[...]) * float(N_HOPS) * 0.5

    hs = [h0[...], h1[...]]
    ts = [t0[...], t1[...]]
    mrs = [mr0[...], mr1[...]]
    relv = rel[...]
    tm = tmat[...]
    w1v = w1[...]
    w2v = w2[...]

    v = items[...]
    mf = 0.0
    kge = 0.0
    l2 = 0.0
    y = jnp.zeros((BB, DIM), jnp.float32)
    for hop in range(N_HOPS):
        h = hs[hop]
        t = ts[hop]
        mr = mrs[hop]
        r = jnp.zeros((BB, RDIM, N_MEMORY, DIM), jnp.float32)
        for k in range(N_REL):
            sel = (mr == k).astype(jnp.float32)[..., None]
            r = r + sel * relv[k][None, None, None, :]
        rh = r * h
        probs = jnp.sum(rh * v[:, None, None, :], axis=-1)
        probs = probs - jnp.max(probs, axis=-1, keepdims=True)
        e = jnp.exp(probs)
        pn = e / jnp.sum(e, axis=-1, keepdims=True)
        o = jnp.sum(t * pn[..., None], axis=2)
        u = jnp.sum(w1v[hop] * w2v[hop][None, :], axis=1)
        att = jnp.maximum(jnp.sum(o * u[None, None, :], axis=-1), 0.0)
        att = att - jnp.max(att, axis=-1, keepdims=True)
        ea = jnp.exp(att)
        an = ea / jnp.sum(ea, axis=-1, keepdims=True)
        o_agg = jnp.sum(o * an[..., None], axis=1)
        v = jnp.dot(v + o_agg, tm, preferred_element_type=jnp.float32)
        y = y + o_agg
        hrt = jnp.sum(rh * t, axis=-1)
        kge = kge + jnp.sum(1.0 / (1.0 + jnp.exp(-hrt)))
        l2 = l2 + jnp.sum(h * h) + jnp.sum(t * t) + jnp.sum(r * r)

    pos_s = jnp.sum(items[...] * y, axis=1)
    neg_s = jnp.sum(neg[...] * y, axis=1)
    d = pos_s - neg_s
    ls = jnp.minimum(d, 0.0) - jnp.log(1.0 + jnp.exp(-jnp.abs(d)))
    mf = jnp.sum(ls)

    out_ref[0] += mf
    out_ref[1] += kge
    out_ref[2] += l2


def _dense_call(items, neg, h0, t0, h1, t1, mr0, mr1, rel, tmat, w1, w2):
    grid = (GSTEPS,)
    bspec = lambda blk: pl.BlockSpec(blk, lambda i: (i,) + (0,) * (len(blk) - 1))
    full = lambda shp: pl.BlockSpec(shp, lambda i: (0,) * len(shp))
    return pl.pallas_call(
        _dense_body,
        grid=grid,
        in_specs=[
            bspec((BB, DIM)),
            bspec((BB, DIM)),
            bspec((BB, RDIM, N_MEMORY, DIM)),
            bspec((BB, RDIM, N_MEMORY, DIM)),
            bspec((BB, RDIM, N_MEMORY, DIM)),
            bspec((BB, RDIM, N_MEMORY, DIM)),
            bspec((BB, RDIM, N_MEMORY)),
            bspec((BB, RDIM, N_MEMORY)),
            full((N_REL, DIM)),
            full((DIM, DIM)),
            full((N_HOPS, DIM, DIM)),
            full((N_HOPS, DIM)),
        ],
        out_specs=pl.BlockSpec(memory_space=pltpu.SMEM),
        out_shape=jax.ShapeDtypeStruct((3,), jnp.float32),
    )(items, neg, h0, t0, h1, t1, mr0, mr1, rel, tmat, w1, w2)


def kernel(pos_items, neg_items, memories_h, memories_r, memories_t,
           entity_emb, relation_emb, transform_matrix, att_w1, att_w2):
    nslots = BATCH * RDIM * N_MEMORY
    idx_all = jnp.concatenate([
        pos_items.astype(jnp.int32),
        neg_items.astype(jnp.int32),
        memories_h.reshape(-1).astype(jnp.int32),
        memories_t.reshape(-1).astype(jnp.int32),
        jnp.zeros((TOT - 2 * BATCH - 2 * N_HOPS * nslots,), jnp.int32),
    ])
    rows = _gather_rows(idx_all, entity_emb)
    items = rows[:BATCH]
    neg = rows[BATCH:2 * BATCH]
    off = 2 * BATCH
    h0 = rows[off:off + nslots].reshape(BATCH, RDIM, N_MEMORY, DIM)
    h1 = rows[off + nslots:off + 2 * nslots].reshape(BATCH, RDIM, N_MEMORY, DIM)
    off += 2 * nslots
    t0 = rows[off:off + nslots].reshape(BATCH, RDIM, N_MEMORY, DIM)
    t1 = rows[off + nslots:off + 2 * nslots].reshape(BATCH, RDIM, N_MEMORY, DIM)

    sums = _dense_call(
        items, neg, h0, t0, h1, t1,
        memories_r[0].astype(jnp.int32), memories_r[1].astype(jnp.int32),
        relation_emb, transform_matrix,
        att_w1, att_w2.reshape(N_HOPS, DIM))

    mf_loss = -sums[0] / BATCH
    kge = sums[1] / (BATCH * RDIM * N_MEMORY)
    return mf_loss - KGE_W * kge + L2_W * sums[2]


# speedup vs baseline: 1.0045x; 1.0045x over previous
import functools

import jax
import jax.numpy as jnp
from jax import lax
from jax.experimental import pallas as pl
from jax.experimental.pallas import tpu as pltpu
from jax.experimental.pallas import tpu_sc as plsc

DIM = 64
N_HOPS = 2
N_MEMORY = 20
N_REL = 9
RDIM = 8
BATCH = 1024
KGE_W = 0.01
L2_W = 1e-5

NC = 2
NS = 16
NW = NC * NS
IDXW = 128
NBUF = 12
AHEAD = 6
NTR = 168
RPW = IDXW * NTR
TOT = NW * RPW


def _sc_gather_body(idx_hbm, table_hbm, out_hbm, idx_v, *bufs_sems):
    rows = bufs_sems[:NBUF]
    gsems = bufs_sems[NBUF:2 * NBUF]
    ssems = bufs_sems[2 * NBUF:3 * NBUF]
    cid = lax.axis_index("c")
    sid = lax.axis_index("s")
    wid = sid * NC + cid
    base = wid * RPW
    pltpu.sync_copy(idx_hbm.at[pl.ds(wid * NTR, NTR)], idx_v)

    def gath(b, tr):
        return pltpu.make_async_copy(table_hbm.at[idx_v.at[tr]], rows[b], gsems[b])

    def stor(b, tr):
        return pltpu.make_async_copy(
            rows[b], out_hbm.at[pl.ds(base + tr * IDXW, IDXW)], ssems[b])

    for b in range(AHEAD):
        gath(b, b).start()

    for k in range(NBUF):
        b = k % NBUF
        gath(b, k).wait()
        stor(b, k).start()
        b2 = (k + AHEAD) % NBUF
        if k >= AHEAD:
            stor(b2, k - AHEAD).wait()
        gath(b2, k + AHEAD).start()

    def body(i, carry):
        for k in range(NBUF):
            tr = i * NBUF + k
            b = k % NBUF
            gath(b, tr).wait()
            stor(b, tr).start()
            b2 = (k + AHEAD) % NBUF
            stor(b2, tr + AHEAD - NBUF).wait()
            gath(b2, tr + AHEAD).start()
        return carry

    lax.fori_loop(1, NTR // NBUF - 1, body, 0)
    for k in range(NBUF):
        tr = NTR - NBUF + k
        b = k % NBUF
        gath(b, tr).wait()
        stor(b, tr).start()
        if k < AHEAD:
            b2 = (k + AHEAD) % NBUF
            stor(b2, tr + AHEAD - NBUF).wait()
            gath(b2, tr + AHEAD).start()
    for k in range(NBUF):
        stor(k, NTR - NBUF + k).wait()


@functools.cache
def _sc_gather_kernel():
    return functools.partial(
        pl.kernel,
        out_type=jax.ShapeDtypeStruct((TOT, DIM), jnp.float32),
        mesh=plsc.VectorSubcoreMesh(core_axis_name="c", subcore_axis_name="s"),
        compiler_params=pltpu.CompilerParams(use_tc_tiling_on_sc=False),
        scratch_types=(
            [pltpu.VMEM((NTR, IDXW), jnp.int32)]
            + [pltpu.VMEM((IDXW, DIM), jnp.float32)] * NBUF
            + [pltpu.SemaphoreType.DMA] * (2 * NBUF)
        ),
    )(_sc_gather_body)


def _gather_rows(idx_all, table):
    return _sc_gather_kernel()(idx_all.reshape(TOT // IDXW, IDXW), table)


BB = 16
GSTEPS = BATCH // BB


def _dense_body(items, neg, h0, t0, h1, t1, mr0, mr1,
                rel, tmat, w1, w2, out_ref):
    step = pl.program_id(0)

    @pl.when(step == 0)
    def _init():
        out_ref[0] = 0.0
        out_ref[1] = 0.0
        out_ref[2] = jnp.sum(tmat[...] * tmat[...]) * float(N_HOPS) * 0.5

    hs = [h0[...], h1[...]]
    ts = [t0[...], t1[...]]
    mrs = [mr0[...], mr1[...]]
    relv = rel[...]
    tm = tmat[...]
    w1v = w1[...]
    w2v = w2[...]

    v = items[...]
    mf = 0.0
    kge = 0.0
    l2 = 0.0
    y = jnp.zeros((BB, DIM), jnp.float32)
    for hop in range(N_HOPS):
        h = hs[hop]
        t = ts[hop]
        mr = mrs[hop]
        r = jnp.zeros((BB, RDIM, N_MEMORY, DIM), jnp.float32)
        for k in range(N_REL):
            sel = (mr == k).astype(jnp.float32)[..., None]
            r = r + sel * relv[k][None, None, None, :]
        rh = r * h
        probs = jnp.sum(rh * v[:, None, None, :], axis=-1)
        probs = probs - jnp.max(probs, axis=-1, keepdims=True)
        e = jnp.exp(probs)
        pn = e / jnp.sum(e, axis=-1, keepdims=True)
        o = jnp.sum(t * pn[..., None], axis=2)
        u = jnp.sum(w1v[hop] * w2v[hop][None, :], axis=1)
        att = jnp.maximum(jnp.sum(o * u[None, None, :], axis=-1), 0.0)
        att = att - jnp.max(att, axis=-1, keepdims=True)
        ea = jnp.exp(att)
        an = ea / jnp.sum(ea, axis=-1, keepdims=True)
        o_agg = jnp.sum(o * an[..., None], axis=1)
        v = jnp.dot(v + o_agg, tm, preferred_element_type=jnp.float32)
        y = y + o_agg
        hrt = jnp.sum(rh * t, axis=-1)
        kge = kge + jnp.sum(1.0 / (1.0 + jnp.exp(-hrt)))
        l2 = l2 + jnp.sum(h * h) + jnp.sum(t * t) + jnp.sum(r * r)

    pos_s = jnp.sum(items[...] * y, axis=1)
    neg_s = jnp.sum(neg[...] * y, axis=1)
    d = pos_s - neg_s
    ls = jnp.minimum(d, 0.0) - jnp.log(1.0 + jnp.exp(-jnp.abs(d)))
    mf = jnp.sum(ls)

    out_ref[0] += mf
    out_ref[1] += kge
    out_ref[2] += l2


def _dense_call(items, neg, h0, t0, h1, t1, mr0, mr1, rel, tmat, w1, w2):
    grid = (GSTEPS,)
    bspec = lambda blk: pl.BlockSpec(blk, lambda i: (i,) + (0,) * (len(blk) - 1))
    full = lambda shp: pl.BlockSpec(shp, lambda i: (0,) * len(shp))
    return pl.pallas_call(
        _dense_body,
        grid=grid,
        in_specs=[
            bspec((BB, DIM)),
            bspec((BB, DIM)),
            bspec((BB, RDIM, N_MEMORY, DIM)),
            bspec((BB, RDIM, N_MEMORY, DIM)),
            bspec((BB, RDIM, N_MEMORY, DIM)),
            bspec((BB, RDIM, N_MEMORY, DIM)),
            bspec((BB, RDIM, N_MEMORY)),
            bspec((BB, RDIM, N_MEMORY)),
            full((N_REL, DIM)),
            full((DIM, DIM)),
            full((N_HOPS, DIM, DIM)),
            full((N_HOPS, DIM)),
        ],
        out_specs=pl.BlockSpec(memory_space=pltpu.SMEM),
        out_shape=jax.ShapeDtypeStruct((3,), jnp.float32),
    )(items, neg, h0, t0, h1, t1, mr0, mr1, rel, tmat, w1, w2)


def kernel(pos_items, neg_items, memories_h, memories_r, memories_t,
           entity_emb, relation_emb, transform_matrix, att_w1, att_w2):
    nslots = BATCH * RDIM * N_MEMORY
    idx_all = jnp.concatenate([
        pos_items.astype(jnp.int32),
        neg_items.astype(jnp.int32),
        memories_h.reshape(-1).astype(jnp.int32),
        memories_t.reshape(-1).astype(jnp.int32),
        jnp.zeros((TOT - 2 * BATCH - 2 * N_HOPS * nslots,), jnp.int32),
    ])
    rows = _gather_rows(idx_all, entity_emb)
    items = rows[:BATCH]
    neg = rows[BATCH:2 * BATCH]
    off = 2 * BATCH
    h0 = rows[off:off + nslots].reshape(BATCH, RDIM, N_MEMORY, DIM)
    h1 = rows[off + nslots:off + 2 * nslots].reshape(BATCH, RDIM, N_MEMORY, DIM)
    off += 2 * nslots
    t0 = rows[off:off + nslots].reshape(BATCH, RDIM, N_MEMORY, DIM)
    t1 = rows[off + nslots:off + 2 * nslots].reshape(BATCH, RDIM, N_MEMORY, DIM)

    sums = _dense_call(
        items, neg, h0, t0, h1, t1,
        memories_r[0].astype(jnp.int32), memories_r[1].astype(jnp.int32),
        relation_emb, transform_matrix,
        att_w1, att_w2.reshape(N_HOPS, DIM))

    mf_loss = -sums[0] / BATCH
    kge = sums[1] / (BATCH * RDIM * N_MEMORY)
    return mf_loss - KGE_W * kge + L2_W * sums[2]
